# initial kernel scaffold (unmeasured)
import jax
import jax.numpy as jnp
from jax import lax
from jax.experimental import pallas as pl
from jax.experimental.pallas import tpu as pltpu


def kernel(
    x,
):
    def body(*refs):
        pass

    out_shape = jax.ShapeDtypeStruct(..., jnp.float32)
    return pl.pallas_call(body, out_shape=out_shape)(...)


# baseline (device time: 49442 ns/iter reference)
import jax
import jax.numpy as jnp
from jax import lax
from jax.experimental import pallas as pl
from jax.experimental.pallas import tpu as pltpu

NZ = 4


def kernel(x):
    m, n = x.shape
    ch = m // NZ
    xb = x.astype(jnp.bfloat16)

    def body(x_ref, out_ref, rs_buf, rs_send, rs_recv, ag_send, ag_recv):
        my_x = lax.axis_index("x")
        my_y = lax.axis_index("y")
        my_z = lax.axis_index("z")

        bsem = pltpu.get_barrier_semaphore()
        for k in range(1, NZ):
            pl.semaphore_signal(
                bsem,
                inc=1,
                device_id=(my_x, my_y, (my_z + k) % NZ),
                device_id_type=pl.DeviceIdType.MESH,
            )
        pl.semaphore_wait(bsem, NZ - 1)

        rs = []
        for k in range(1, NZ):
            dst = (my_z + k) % NZ
            rdma = pltpu.make_async_remote_copy(
                src_ref=x_ref.at[pl.ds(dst * ch, ch), :],
                dst_ref=rs_buf.at[k - 1],
                send_sem=rs_send.at[k - 1],
                recv_sem=rs_recv.at[k - 1],
                device_id=(my_x, my_y, dst),
                device_id_type=pl.DeviceIdType.MESH,
            )
            rdma.start()
            rs.append(rdma)
        for rdma in rs:
            rdma.wait()

        acc = x_ref[pl.ds(my_z * ch, ch), :].astype(jnp.float32)
        for k in range(1, NZ):
            acc += rs_buf[k - 1, :, :].astype(jnp.float32)
        out_ref[pl.ds(my_z * ch, ch), :] = acc.astype(jnp.bfloat16)

        ag = []
        for k in range(1, NZ):
            dst = (my_z + k) % NZ
            rdma = pltpu.make_async_remote_copy(
                src_ref=out_ref.at[pl.ds(my_z * ch, ch), :],
                dst_ref=out_ref.at[pl.ds(my_z * ch, ch), :],
                send_sem=ag_send.at[k - 1],
                recv_sem=ag_recv.at[k - 1],
                device_id=(my_x, my_y, dst),
                device_id_type=pl.DeviceIdType.MESH,
            )
            rdma.start()
            ag.append(rdma)
        for rdma in ag:
            rdma.wait()

    return pl.pallas_call(
        body,
        out_shape=jax.ShapeDtypeStruct((m, n), jnp.bfloat16),
        in_specs=[pl.BlockSpec(memory_space=pltpu.VMEM)],
        out_specs=pl.BlockSpec(memory_space=pltpu.VMEM),
        scratch_shapes=[
            pltpu.VMEM((NZ - 1, ch, n), jnp.bfloat16),
            pltpu.SemaphoreType.DMA((NZ - 1,)),
            pltpu.SemaphoreType.DMA((NZ - 1,)),
            pltpu.SemaphoreType.DMA((NZ - 1,)),
            pltpu.SemaphoreType.DMA((NZ - 1,)),
        ],
        compiler_params=pltpu.CompilerParams(collective_id=0),
    )(xb)
